# baseline (speedup 1.0000x reference)
import functools

import jax
import jax.numpy as jnp
from jax import lax
from jax.experimental import pallas as pl
from jax.experimental.pallas import tpu as pltpu
from jax.experimental.pallas import tpu_sc as plsc

_E = 8
_R = 16
_SCALING = 32.0 / 16.0
_TB = 512

_BF16_DOT = jax.lax.Precision.DEFAULT


def _router_block(x_ref, rw_ref, logits_ref, logits_t_ref):
    x = x_ref[0]
    rw = rw_ref[...]
    logits_ref[...] = jax.lax.dot_general(
        x, rw, (((1,), (1,)), ((), ())),
        precision=_BF16_DOT, preferred_element_type=jnp.float32)
    logits_t_ref[...] = jax.lax.dot_general(
        rw, x, (((1,), (1,)), ((), ())),
        precision=_BF16_DOT, preferred_element_type=jnp.float32)


def _make_sc_mask(t):
    info = plsc.get_sparse_core_info()
    nc, ns = info.num_cores, info.num_subcores
    tpw = 128
    nact = t // tpw
    mesh = plsc.VectorSubcoreMesh(core_axis_name="c", subcore_axis_name="s")

    @functools.partial(
        pl.kernel, mesh=mesh,
        out_type=jax.ShapeDtypeStruct((_E, t), jnp.float32),
        scratch_types=[
            pltpu.VMEM((_E, tpw), jnp.float32),
            pltpu.VMEM((_E, tpw), jnp.float32),
        ],
    )
    def sc_mask(logits_t_hbm, out_hbm, lv, mv):
        wid = lax.axis_index("s") * nc + lax.axis_index("c")
        base = jnp.minimum(wid, nact - 1) * tpw
        pltpu.sync_copy(logits_t_hbm.at[:, pl.ds(base, tpw)], lv)
        lo = jnp.full((16,), -3.0e38, jnp.float32)
        for chunk in range(tpw // 16):
            sl = pl.ds(chunk * 16, 16)
            le = [lv[e, sl] for e in range(_E)]
            m1 = le[0]
            for e in range(1, _E):
                m1 = jnp.maximum(m1, le[e])
            i1 = jnp.full((16,), float(_E), jnp.float32)
            for e in range(_E - 1, -1, -1):
                ev = jnp.full((16,), float(e), jnp.float32)
                i1 = jnp.where(le[e] == m1, ev, i1)
            m2 = lo
            for e in range(_E):
                ev = jnp.full((16,), float(e), jnp.float32)
                l2e = jnp.where(i1 == ev, lo, le[e])
                m2 = jnp.maximum(m2, l2e)
            i2 = jnp.full((16,), float(_E), jnp.float32)
            for e in range(_E - 1, -1, -1):
                ev = jnp.full((16,), float(e), jnp.float32)
                keep = jnp.where(i1 == ev, i2, ev)
                i2 = jnp.where(le[e] == m2, keep, i2)
            one = jnp.full((16,), 1.0, jnp.float32)
            zero = jnp.full((16,), 0.0, jnp.float32)
            for e in range(_E):
                ev = jnp.full((16,), float(e), jnp.float32)
                hit1 = jnp.where(i1 == ev, one, zero)
                hit2 = jnp.where(i2 == ev, one, zero)
                mv[e, sl] = jnp.maximum(hit1, hit2)
        @pl.when(wid < nact)
        def _():
            pltpu.sync_copy(mv, out_hbm.at[:, pl.ds(base, tpw)])

    return sc_mask


def _moe_block(x_ref, mask_t_ref, w1_ref, w3_ref, w2_ref, a_ref, b_ref,
               out_ref):
    x = x_ref[0]
    mt = mask_t_ref[...]

    expand = (jax.lax.broadcasted_iota(jnp.int32, (_E, _E * _R), 1) // _R
              == jax.lax.broadcasted_iota(jnp.int32, (_E, _E * _R), 0)
              ).astype(jnp.float32)
    mask128 = jax.lax.dot_general(mt, expand, (((0,), (0,)), ((), ())),
                                  precision=_BF16_DOT)

    h1 = jax.lax.dot_general(x, w1_ref[...], (((1,), (1,)), ((), ())),
                             precision=_BF16_DOT)
    h3 = jax.lax.dot_general(x, w3_ref[...], (((1,), (1,)), ((), ())),
                             precision=_BF16_DOT)
    h = jax.nn.silu(h1) * h3
    ex = jax.lax.dot_general(h, w2_ref[...], (((1,), (1,)), ((), ())),
                             precision=_BF16_DOT)

    z = jax.lax.dot_general(x, a_ref[...], (((1,), (1,)), ((), ())),
                            precision=_BF16_DOT)
    zm = z * mask128
    lora = jax.lax.dot_general(zm, b_ref[...], (((1,), (0,)), ((), ())),
                               precision=_BF16_DOT)

    out_ref[0] = ex + _SCALING * lora


@functools.partial(jax.jit, static_argnames=())
def kernel(hidden_states, router_w, w1, w2, w3, lora_A, lora_B):
    bs, sl, hd = hidden_states.shape
    t = bs * sl
    ffn = w1.shape[0]
    e, r, d = lora_A.shape

    a_cat = lora_A.reshape(e * r, d)
    b_cat = lora_B.transpose(0, 2, 1).reshape(e * r, d)

    const = lambda i: (0, 0)

    logits, logits_t = pl.pallas_call(
        _router_block,
        grid=(t // _TB,),
        in_specs=[
            pl.BlockSpec((1, _TB, hd), lambda i: (0, i, 0)),
            pl.BlockSpec((_E, hd), const),
        ],
        out_specs=[
            pl.BlockSpec((_TB, _E), lambda i: (i, 0)),
            pl.BlockSpec((_E, _TB), lambda i: (0, i)),
        ],
        out_shape=[
            jax.ShapeDtypeStruct((t, _E), jnp.float32),
            jax.ShapeDtypeStruct((_E, t), jnp.float32),
        ],
        compiler_params=pltpu.CompilerParams(
            dimension_semantics=("arbitrary",)),
    )(hidden_states, router_w)

    mask_t = _make_sc_mask(t)(logits_t)

    final = pl.pallas_call(
        _moe_block,
        grid=(t // _TB,),
        in_specs=[
            pl.BlockSpec((1, _TB, hd), lambda i: (0, i, 0)),
            pl.BlockSpec((_E, _TB), lambda i: (0, i)),
            pl.BlockSpec((ffn, hd), const),
            pl.BlockSpec((ffn, hd), const),
            pl.BlockSpec((hd, ffn), const),
            pl.BlockSpec((e * r, hd), const),
            pl.BlockSpec((e * r, hd), const),
        ],
        out_specs=pl.BlockSpec((1, _TB, hd), lambda i: (0, i, 0)),
        out_shape=jax.ShapeDtypeStruct((bs, sl, hd), jnp.float32),
        compiler_params=pltpu.CompilerParams(
            dimension_semantics=("arbitrary",)),
    )(hidden_states, mask_t, w1, w3, w2, a_cat, b_cat)

    return final, logits


# speedup vs baseline: 1.4510x; 1.4510x over previous
import functools

import jax
import jax.numpy as jnp
from jax.experimental import pallas as pl
from jax.experimental.pallas import tpu as pltpu

_E = 8
_TOPK = 2
_R = 16
_SCALING = 32.0 / 16.0
_TB = 1024

_BF16_DOT = jax.lax.Precision.DEFAULT


def _moe_block(x_ref, rw_ref, w1_ref, w3_ref, w2_ref, a_ref, b_ref,
               out_ref, logits_ref):
    x = x_ref[0]

    logits = jax.lax.dot_general(
        x, rw_ref[...], (((1,), (1,)), ((), ())),
        precision=_BF16_DOT, preferred_element_type=jnp.float32)
    logits_ref[...] = logits

    tb = logits.shape[0]
    idx = jax.lax.broadcasted_iota(jnp.int32, (tb, _E), 1)
    m1 = jnp.max(logits, axis=1, keepdims=True)
    i1 = jnp.min(jnp.where(logits == m1, idx, _E), axis=1, keepdims=True)
    l2 = jnp.where(idx == i1, -jnp.inf, logits)
    m2 = jnp.max(l2, axis=1, keepdims=True)
    i2 = jnp.min(jnp.where(l2 == m2, idx, _E), axis=1, keepdims=True)

    col_e = jax.lax.broadcasted_iota(jnp.int32, (tb, _E * _R), 1) // _R
    mask = (col_e == i1) | (col_e == i2)

    h1 = jax.lax.dot_general(x, w1_ref[...], (((1,), (1,)), ((), ())),
                             precision=_BF16_DOT)
    h3 = jax.lax.dot_general(x, w3_ref[...], (((1,), (1,)), ((), ())),
                             precision=_BF16_DOT)
    h = jax.nn.silu(h1) * h3
    ex = jax.lax.dot_general(h, w2_ref[...], (((1,), (1,)), ((), ())),
                             precision=_BF16_DOT)

    z = jax.lax.dot_general(x, a_ref[...], (((1,), (1,)), ((), ())),
                            precision=_BF16_DOT)
    zm = jnp.where(mask, z, 0.0)
    lora = jax.lax.dot_general(zm, b_ref[...], (((1,), (0,)), ((), ())),
                               precision=_BF16_DOT)

    out_ref[0] = ex + _SCALING * lora


@functools.partial(jax.jit, static_argnames=())
def kernel(hidden_states, router_w, w1, w2, w3, lora_A, lora_B):
    bs, sl, hd = hidden_states.shape
    t = bs * sl
    ffn = w1.shape[0]
    e, r, d = lora_A.shape

    a_cat = lora_A.reshape(e * r, d)
    b_cat = lora_B.transpose(0, 2, 1).reshape(e * r, d)

    grid = (t // _TB,)
    const = lambda i: (0, 0)
    final, logits = pl.pallas_call(
        _moe_block,
        grid=grid,
        in_specs=[
            pl.BlockSpec((1, _TB, hd), lambda i: (0, i, 0)),
            pl.BlockSpec((_E, hd), const),
            pl.BlockSpec((ffn, hd), const),
            pl.BlockSpec((ffn, hd), const),
            pl.BlockSpec((hd, ffn), const),
            pl.BlockSpec((e * r, hd), const),
            pl.BlockSpec((e * r, hd), const),
        ],
        out_specs=[
            pl.BlockSpec((1, _TB, hd), lambda i: (0, i, 0)),
            pl.BlockSpec((_TB, _E), lambda i: (i, 0)),
        ],
        out_shape=[
            jax.ShapeDtypeStruct((bs, sl, hd), jnp.float32),
            jax.ShapeDtypeStruct((t, _E), jnp.float32),
        ],
        compiler_params=pltpu.CompilerParams(
            dimension_semantics=("arbitrary",)),
    )(hidden_states, router_w, w1, w3, w2, a_cat, b_cat)

    return final, logits


# speedup vs baseline: 1.4978x; 1.0323x over previous
import functools

import jax
import jax.numpy as jnp
from jax.experimental import pallas as pl
from jax.experimental.pallas import tpu as pltpu

_E = 8
_TOPK = 2
_R = 16
_SCALING = 32.0 / 16.0
_TB = 512

_BF16_DOT = jax.lax.Precision.DEFAULT


def _moe_block(x_ref, rw_ref, w1_ref, w3_ref, w2_ref, a_ref, b_ref,
               out_ref, logits_ref):
    x = x_ref[0]

    logits = jax.lax.dot_general(
        x, rw_ref[...], (((1,), (1,)), ((), ())),
        precision=_BF16_DOT, preferred_element_type=jnp.float32)
    logits_ref[...] = logits

    tb = logits.shape[0]
    idx = jax.lax.broadcasted_iota(jnp.int32, (tb, _E), 1)
    m1 = jnp.max(logits, axis=1, keepdims=True)
    i1 = jnp.min(jnp.where(logits == m1, idx, _E), axis=1, keepdims=True)
    l2 = jnp.where(idx == i1, -jnp.inf, logits)
    m2 = jnp.max(l2, axis=1, keepdims=True)
    i2 = jnp.min(jnp.where(l2 == m2, idx, _E), axis=1, keepdims=True)

    col_e = jax.lax.broadcasted_iota(jnp.int32, (tb, _E * _R), 1) // _R
    mask = (col_e == i1) | (col_e == i2)

    h1 = jax.lax.dot_general(x, w1_ref[...], (((1,), (1,)), ((), ())),
                             precision=_BF16_DOT)
    h3 = jax.lax.dot_general(x, w3_ref[...], (((1,), (1,)), ((), ())),
                             precision=_BF16_DOT)
    h = jax.nn.silu(h1) * h3
    ex = jax.lax.dot_general(h, w2_ref[...], (((1,), (1,)), ((), ())),
                             precision=_BF16_DOT)

    z = jax.lax.dot_general(x, a_ref[...], (((1,), (1,)), ((), ())),
                            precision=_BF16_DOT)
    zm = jnp.where(mask, z, 0.0)
    lora = jax.lax.dot_general(zm, b_ref[...], (((1,), (0,)), ((), ())),
                               precision=_BF16_DOT)

    out_ref[0] = ex + _SCALING * lora


@functools.partial(jax.jit, static_argnames=())
def kernel(hidden_states, router_w, w1, w2, w3, lora_A, lora_B):
    bs, sl, hd = hidden_states.shape
    t = bs * sl
    ffn = w1.shape[0]
    e, r, d = lora_A.shape

    a_cat = lora_A.reshape(e * r, d)
    b_cat = lora_B.transpose(0, 2, 1).reshape(e * r, d)

    grid = (t // _TB,)
    const = lambda i: (0, 0)
    final, logits = pl.pallas_call(
        _moe_block,
        grid=grid,
        in_specs=[
            pl.BlockSpec((1, _TB, hd), lambda i: (0, i, 0)),
            pl.BlockSpec((_E, hd), const),
            pl.BlockSpec((ffn, hd), const),
            pl.BlockSpec((ffn, hd), const),
            pl.BlockSpec((hd, ffn), const),
            pl.BlockSpec((e * r, hd), const),
            pl.BlockSpec((e * r, hd), const),
        ],
        out_specs=[
            pl.BlockSpec((1, _TB, hd), lambda i: (0, i, 0)),
            pl.BlockSpec((_TB, _E), lambda i: (i, 0)),
        ],
        out_shape=[
            jax.ShapeDtypeStruct((bs, sl, hd), jnp.float32),
            jax.ShapeDtypeStruct((t, _E), jnp.float32),
        ],
        compiler_params=pltpu.CompilerParams(
            dimension_semantics=("arbitrary",)),
    )(hidden_states, router_w, w1, w3, w2, a_cat, b_cat)

    return final, logits
